# baseline (speedup 1.0000x reference)
import functools

import jax
import jax.numpy as jnp
from jax import lax
from jax.experimental import pallas as pl
from jax.experimental.pallas import tpu as pltpu
from jax.experimental.pallas import tpu_sc as plsc

B = 4096
S = 50
D = 64
P = 20
T = P + S

NC = 2
NS = 16
NW = NC * NS
NB = B // NW
G = 8
NG = NB // G
NGH = NG // 2


def _soft_embed(ids_hbm, table_hbm, learned_hbm, out_hbm,
                idx_v, obuf0, obuf1, sem0, sem1):
    wid = lax.axis_index("s") * NC + lax.axis_index("c")
    b0 = wid * NB
    pltpu.sync_copy(ids_hbm.at[pl.ds(b0, NB)], idx_v)
    for g in range(G):
        pltpu.sync_copy(learned_hbm, obuf0.at[g, pl.ds(0, P)])
        pltpu.sync_copy(learned_hbm, obuf1.at[g, pl.ds(0, P)])

    def fire(og, buf, sem):
        for g in range(G):
            pltpu.async_copy(
                table_hbm.at[idx_v.at[og * G + g]], buf.at[g, pl.ds(P, S)], sem
            )

    def drain(og, buf, sem):
        for g in range(G):
            pltpu.make_async_copy(
                table_hbm.at[idx_v.at[og * G + g]], buf.at[g, pl.ds(P, S)], sem
            ).wait()

    fire(0, obuf0, sem0)
    fire(1, obuf1, sem1)

    def body(j, carry):
        og = 2 * j
        drain(og, obuf0, sem0)
        pltpu.sync_copy(obuf0, out_hbm.at[pl.ds(b0 + og * G, G)])

        @pl.when(j < NGH - 1)
        def _():
            fire(og + 2, obuf0, sem0)

        drain(og + 1, obuf1, sem1)
        pltpu.sync_copy(obuf1, out_hbm.at[pl.ds(b0 + (og + 1) * G, G)])

        @pl.when(j < NGH - 1)
        def _():
            fire(og + 3, obuf1, sem1)

        return carry

    lax.fori_loop(0, NGH, body, 0)


def kernel(input_ids, table, learned):
    mesh = plsc.VectorSubcoreMesh(core_axis_name="c", subcore_axis_name="s")
    run = functools.partial(
        pl.kernel,
        mesh=mesh,
        out_type=jax.ShapeDtypeStruct((B, T, D), jnp.float32),
        scratch_types=[
            pltpu.VMEM((NB, S), jnp.int32),
            pltpu.VMEM((G, T, D), jnp.float32),
            pltpu.VMEM((G, T, D), jnp.float32),
            pltpu.SemaphoreType.DMA,
            pltpu.SemaphoreType.DMA,
        ],
        compiler_params=pltpu.CompilerParams(use_tc_tiling_on_sc=False),
    )(_soft_embed)
    return run(input_ids, table, learned)
